# Initial kernel scaffold; baseline (speedup 1.0000x reference)
import functools

import jax
import jax.numpy as jnp
from jax import lax
from jax.experimental import pallas as pl
from jax.experimental.pallas import tpu as pltpu
from jax.experimental.pallas import tpu_sc as plsc

V = 1000000
D = 64
B = 16384
F = 26
N = B * F

_info = plsc.get_sparse_core_info()
NC, NS, L = _info.num_cores, _info.num_subcores, _info.num_lanes
NW = NC * NS
NPW = N // NW
CH = 128
NCH = NPW // CH


def _body(idx2d, center, offset, lo, hi, idx_v, c_v, o_v, lo_v, hi_v, sem):
    wid = lax.axis_index("s") * NC + lax.axis_index("c")
    row0 = wid * NCH
    out0 = wid * NPW
    pltpu.sync_copy(idx2d.at[pl.ds(row0, NCH)], idx_v)

    def chunk(j, carry):
        cp_c = pltpu.async_copy(center.at[idx_v.at[j]], c_v, sem)
        cp_o = pltpu.async_copy(offset.at[idx_v.at[j]], o_v, sem)
        cp_c.wait()
        cp_o.wait()

        def row(r, carry2):
            for s in range(D // L):
                sl = pl.ds(s * L, L)
                c = c_v[r, sl]
                x = o_v[r, sl]
                t = jnp.exp(-jnp.abs(x))
                z = t / (t + 2.0)
                z2 = z * z
                sp = jnp.maximum(x, 0.0) + z * (
                    2.0
                    + z2
                    * (
                        2.0 / 3.0
                        + z2 * (2.0 / 5.0 + z2 * (2.0 / 7.0 + z2 * (2.0 / 9.0)))
                    )
                )
                lo_v[r, sl] = c - sp
                hi_v[r, sl] = c + sp
            return carry2

        lax.fori_loop(0, CH, row, 0)
        pltpu.sync_copy(lo_v, lo.at[pl.ds(out0 + j * CH, CH)])
        pltpu.sync_copy(hi_v, hi.at[pl.ds(out0 + j * CH, CH)])
        return carry

    lax.fori_loop(0, NCH, chunk, 0)


def _run(idx2d, center, offset):
    mesh = plsc.VectorSubcoreMesh(core_axis_name="c", subcore_axis_name="s")
    f = functools.partial(
        pl.kernel,
        mesh=mesh,
        out_type=[
            jax.ShapeDtypeStruct((N, D), jnp.float32),
            jax.ShapeDtypeStruct((N, D), jnp.float32),
        ],
        scratch_types=[
            pltpu.VMEM((NCH, CH), jnp.int32),
            pltpu.VMEM((CH, D), jnp.float32),
            pltpu.VMEM((CH, D), jnp.float32),
            pltpu.VMEM((CH, D), jnp.float32),
            pltpu.VMEM((CH, D), jnp.float32),
            pltpu.SemaphoreType.DMA,
        ],
    )(_body)
    return f(idx2d, center, offset)


def kernel(idx, center, offset):
    idx2d = idx.astype(jnp.int32).reshape(N // CH, CH)
    lo, hi = _run(idx2d, center, offset)
    return (lo.reshape(B, F, D), hi.reshape(B, F, D))


# speedup vs baseline: 1.0282x; 1.0282x over previous
import functools

import jax
import jax.numpy as jnp
from jax import lax
from jax.experimental import pallas as pl
from jax.experimental.pallas import tpu as pltpu
from jax.experimental.pallas import tpu_sc as plsc

V = 1000000
D = 64
B = 16384
F = 26
N = B * F

_info = plsc.get_sparse_core_info()
NC, NS, L = _info.num_cores, _info.num_subcores, _info.num_lanes
NW = NC * NS
BPW = B // NW
CHB = 4
CH = CHB * F
NCH = BPW // CHB


def _softplus(x):
    t = jnp.exp(-jnp.abs(x))
    z = t / (t + 2.0)
    z2 = z * z
    return jnp.maximum(x, 0.0) + z * (
        2.0
        + z2 * (2.0 / 3.0 + z2 * (2.0 / 5.0 + z2 * (2.0 / 7.0 + z2 * (2.0 / 9.0))))
    )


def _body(idx_hbm, center, offset, lo, hi, idx_s, idx_v, c2, o2, lo3, hi3,
          gsem0, gsem1, osem0, osem1):
    wid = lax.axis_index("s") * NC + lax.axis_index("c")
    wb = wid * BPW
    pltpu.sync_copy(idx_hbm.at[pl.ds(wb, BPW)], idx_s)

    def repack(q, carry):
        for p in range(CHB):
            r = q * CHB + p
            idx_v[q, pl.ds(F * p, L)] = idx_s[r, pl.ds(0, L)]
            idx_v[q, pl.ds(F * p + F - L, L)] = idx_s[r, pl.ds(F - L, L)]
        return carry

    lax.fori_loop(0, NCH, repack, 0)

    gsems = (gsem0, gsem1)
    osems = (osem0, osem1)

    def fire_gather(j, b):
        pltpu.async_copy(center.at[idx_v.at[j]], c2.at[b], gsems[b])
        pltpu.async_copy(offset.at[idx_v.at[j]], o2.at[b], gsems[b])

    def wait_gather(b):
        pltpu.make_async_copy(center.at[idx_v.at[0]], c2.at[b], gsems[b]).wait()
        pltpu.make_async_copy(offset.at[idx_v.at[0]], o2.at[b], gsems[b]).wait()

    def fire_out(j, b):
        dst = pl.ds(wb + j * CHB, CHB)
        pltpu.async_copy(lo3.at[b], lo.at[dst], osems[b])
        pltpu.async_copy(hi3.at[b], hi.at[dst], osems[b])

    def wait_out(b):
        dst = pl.ds(wb, CHB)
        pltpu.make_async_copy(lo3.at[b], lo.at[dst], osems[b]).wait()
        pltpu.make_async_copy(hi3.at[b], hi.at[dst], osems[b]).wait()

    def compute(b):
        for bb in range(CHB):
            def row(f, carry):
                r = bb * F + f
                for s in range(D // L):
                    sl = pl.ds(s * L, L)
                    c = c2[b, r, sl]
                    sp = _softplus(o2[b, r, sl])
                    lo3[b, bb, f, sl] = c - sp
                    hi3[b, bb, f, sl] = c + sp
                return carry

            lax.fori_loop(0, F, row, 0)

    fire_gather(0, 0)

    def step(j2, carry):
        for ph in range(2):
            j = j2 * 2 + ph

            @pl.when(j + 1 < NCH)
            def _():
                fire_gather(j + 1, 1 - ph)

            wait_gather(ph)

            @pl.when(j >= 2)
            def _():
                wait_out(ph)

            compute(ph)
            fire_out(j, ph)
        return carry

    lax.fori_loop(0, NCH // 2, step, 0)
    wait_out(0)
    wait_out(1)


def _run(idx, center, offset):
    mesh = plsc.VectorSubcoreMesh(core_axis_name="c", subcore_axis_name="s")
    f = functools.partial(
        pl.kernel,
        mesh=mesh,
        out_type=[
            jax.ShapeDtypeStruct((B, F, D), jnp.float32),
            jax.ShapeDtypeStruct((B, F, D), jnp.float32),
        ],
        scratch_types=[
            pltpu.VMEM((BPW, F), jnp.int32),
            pltpu.VMEM((NCH, CH), jnp.int32),
            pltpu.VMEM((2, CH, D), jnp.float32),
            pltpu.VMEM((2, CH, D), jnp.float32),
            pltpu.VMEM((2, CHB, F, D), jnp.float32),
            pltpu.VMEM((2, CHB, F, D), jnp.float32),
            pltpu.SemaphoreType.DMA,
            pltpu.SemaphoreType.DMA,
            pltpu.SemaphoreType.DMA,
            pltpu.SemaphoreType.DMA,
        ],
        compiler_params=pltpu.CompilerParams(use_tc_tiling_on_sc=False),
    )(_body)
    return f(idx, center, offset)


def kernel(idx, center, offset):
    lo, hi = _run(idx.astype(jnp.int32), center, offset)
    return (lo, hi)
